# baseline (speedup 1.0000x reference)
import jax
import jax.numpy as jnp
from jax.experimental import pallas as pl
from jax.experimental.pallas import tpu as pltpu

NUM_LAYERS = 4
D = 2048
BATCH = 4096
TILE_B = 256
NB = BATCH // TILE_B


def _layer_kernel(x_ref, c0_ref, w_ref, b_ref, h_ref, c1_ref, u_scr, x_scr):
    u = u_scr[...]
    xp = x_scr[...]
    f = jax.nn.sigmoid(u[:, D:2 * D] + b_ref[0, :])
    r = jax.nn.sigmoid(u[:, 2 * D:] + b_ref[1, :])
    c1 = f * c0_ref[...] + (1.0 - f) * u[:, :D]
    h = r * jnp.tanh(c1) + (1.0 - r) * xp.astype(jnp.float32)
    h_ref[...] = h.astype(h_ref.dtype)
    c1_ref[...] = c1
    xb = x_ref[...]
    u_scr[...] = jnp.dot(xb, w_ref[...], preferred_element_type=jnp.float32)
    x_scr[...] = xb


def _layer(x_bf16, c0, w_bf16, b2, h_dtype):
    return pl.pallas_call(
        _layer_kernel,
        grid=(NB + 1,),
        in_specs=[
            pl.BlockSpec((TILE_B, D), lambda i: (jnp.minimum(i, NB - 1), 0)),
            pl.BlockSpec((TILE_B, D), lambda i: (jnp.maximum(i - 1, 0), 0)),
            pl.BlockSpec((D, 3 * D), lambda i: (0, 0)),
            pl.BlockSpec((2, D), lambda i: (0, 0)),
        ],
        out_specs=[
            pl.BlockSpec((TILE_B, D), lambda i: (jnp.maximum(i - 1, 0), 0)),
            pl.BlockSpec((TILE_B, D), lambda i: (jnp.maximum(i - 1, 0), 0)),
        ],
        out_shape=[
            jax.ShapeDtypeStruct((BATCH, D), h_dtype),
            jax.ShapeDtypeStruct((BATCH, D), jnp.float32),
        ],
        scratch_shapes=[
            pltpu.VMEM((TILE_B, 3 * D), jnp.float32),
            pltpu.VMEM((TILE_B, D), jnp.bfloat16),
        ],
        compiler_params=pltpu.CompilerParams(
            dimension_semantics=("arbitrary",),
        ),
    )(x_bf16, c0, w_bf16, b2)


def kernel(input, c_0, W0, b0, W1, b1, W2, b2, W3, b3):
    Ws = [W0, W1, W2, W3]
    bs = [b0, b1, b2, b3]
    h = input.astype(jnp.bfloat16)
    c1_list = []
    for i in range(NUM_LAYERS):
        h_dtype = jnp.float32 if i == NUM_LAYERS - 1 else jnp.bfloat16
        h, c1 = _layer(h, c_0[i], Ws[i].astype(jnp.bfloat16),
                       bs[i].reshape(2, D), h_dtype)
        c1_list.append(c1)
    return (h, jnp.stack(c1_list))


# speedup vs baseline: 1.3201x; 1.3201x over previous
import jax
import jax.numpy as jnp
from jax.experimental import pallas as pl
from jax.experimental.pallas import tpu as pltpu

NUM_LAYERS = 4
D = 2048
BATCH = 4096
TILE_B = 256
NB = BATCH // TILE_B


def _make_layer_kernel(has_cbuf):
    def _layer_kernel(*refs):
        if has_cbuf:
            x_ref, c0_ref, w_ref, b_ref, _cbuf_ref, h_ref, c1_ref = refs
        else:
            x_ref, c0_ref, w_ref, b_ref, h_ref, c1_ref = refs
        xb = x_ref[...]
        u = jnp.dot(xb, w_ref[...], preferred_element_type=jnp.float32)
        f = 0.5 * jnp.tanh(0.5 * (u[:, D:2 * D] + b_ref[0, :])) + 0.5
        r = 0.5 * jnp.tanh(0.5 * (u[:, 2 * D:] + b_ref[1, :])) + 0.5
        c1 = f * c0_ref[...] + (1.0 - f) * u[:, :D]
        h = r * jnp.tanh(c1) + (1.0 - r) * xb.astype(jnp.float32)
        h_ref[...] = h.astype(h_ref.dtype)
        c1_ref[0] = c1
    return _layer_kernel


def _layer(layer_idx, x_bf16, c0, w_bf16, b2, cbuf, h_dtype):
    in_specs = [
        pl.BlockSpec((TILE_B, D), lambda i: (i, 0)),
        pl.BlockSpec((TILE_B, D), lambda i: (i, 0)),
        pl.BlockSpec((D, 3 * D), lambda i: (0, 0)),
        pl.BlockSpec((2, D), lambda i: (0, 0)),
    ]
    args = [x_bf16, c0, w_bf16, b2]
    aliases = {}
    if cbuf is not None:
        in_specs.append(pl.BlockSpec((1, 8, 128), lambda i: (0, 0, 0)))
        args.append(cbuf)
        aliases = {4: 1}
    return pl.pallas_call(
        _make_layer_kernel(cbuf is not None),
        grid=(NB,),
        in_specs=in_specs,
        out_specs=[
            pl.BlockSpec((TILE_B, D), lambda i: (i, 0)),
            pl.BlockSpec((1, TILE_B, D), lambda i: (layer_idx, i, 0)),
        ],
        out_shape=[
            jax.ShapeDtypeStruct((BATCH, D), h_dtype),
            jax.ShapeDtypeStruct((NUM_LAYERS, BATCH, D), jnp.float32),
        ],
        input_output_aliases=aliases,
        compiler_params=pltpu.CompilerParams(
            dimension_semantics=("arbitrary",),
        ),
    )(*args)


def kernel(input, c_0, W0, b0, W1, b1, W2, b2, W3, b3):
    Ws = [W0, W1, W2, W3]
    bs = [b0, b1, b2, b3]
    h = input.astype(jnp.bfloat16)
    cbuf = None
    for i in range(NUM_LAYERS):
        h_dtype = jnp.float32 if i == NUM_LAYERS - 1 else jnp.bfloat16
        h, cbuf = _layer(i, h, c_0[i], Ws[i].astype(jnp.bfloat16),
                         bs[i].reshape(2, D), cbuf, h_dtype)
    return (h, cbuf)
